# baseline (speedup 1.0000x reference)
import jax
import jax.numpy as jnp
from jax.experimental import pallas as pl
from jax.experimental.pallas import tpu as pltpu

B = 4096
IN = 3072
HID = 256
OUT = 10
E = 2
K = 2
LOSS_COEF = 0.01

BLK = 1024
LANES = 128


def _moe_kernel(scal_ref, x_ref, w1a_ref, w1b_ref, wg_ref, b1_ref, w2_ref,
                b2_ref, y_ref, loss_ref, imp_ref):
    i = pl.program_id(0)
    nsteps = pl.num_programs(0)

    tau1 = scal_ref[0]
    tau2 = scal_ref[1]
    bg0 = scal_ref[2]
    bg1 = scal_ref[3]

    xb = x_ref[...].astype(jnp.bfloat16)
    pre0 = jnp.dot(xb, w1a_ref[...],
                   preferred_element_type=jnp.float32)
    pre1 = jnp.dot(xb, w1b_ref[...],
                   preferred_element_type=jnp.float32)
    gl = jnp.dot(xb, wg_ref[...],
                 preferred_element_type=jnp.float32)

    h0 = jnp.tanh(pre0 + b1_ref[0:1, :])
    h1 = jnp.tanh(pre1 + b1_ref[1:2, :])
    l0 = gl[:, 0:1] + bg0
    l1 = gl[:, 1:2] + bg1

    s0 = jax.nn.sigmoid(l0 - l1)
    s1 = jax.nn.sigmoid(l1 - l0)
    r0 = jax.nn.sigmoid((s1 - s0) / tau1)
    r1 = jax.nn.sigmoid((s0 - s1) / tau1)
    a0 = jax.nn.sigmoid((K + 0.5 - (1.0 + r0)) / tau2)
    a1 = jax.nn.sigmoid((K + 0.5 - (1.0 + r1)) / tau2)
    g0 = a0 * s0
    g1 = a1 * s1

    hs = jnp.concatenate([h0 * g0, h1 * g1], axis=1)
    out = jnp.dot(hs.astype(jnp.bfloat16), w2_ref[...],
                  preferred_element_type=jnp.float32)
    out = out + g0 * b2_ref[0:1, :] + g1 * b2_ref[1:2, :]
    y_ref[...] = out[:, :OUT]

    p0 = jnp.sum(g0)
    p1 = jnp.sum(g1)
    t0 = jnp.where(i == 0, 0.0, imp_ref[0]) + p0
    t1 = jnp.where(i == 0, 0.0, imp_ref[1]) + p1
    imp_ref[0] = t0
    imp_ref[1] = t1

    @pl.when(i == nsteps - 1)
    def _():
        m = (t0 + t1) * 0.5
        var = (t0 - m) ** 2 + (t1 - m) ** 2
        loss_ref[0, 0] = var / (m * m + 1e-10) * LOSS_COEF


@jax.jit
def _moe(x, Wg, bg, W1, b1, W2, b2, tau1, tau2):
    xf = x.reshape(B, IN)
    w1a = W1[0].astype(jnp.bfloat16)
    w1b = W1[1].astype(jnp.bfloat16)
    wgp = jnp.pad(Wg, ((0, 0), (0, LANES - E))).astype(jnp.bfloat16)
    w2c = jnp.pad(W2.reshape(E * HID, OUT),
                  ((0, 0), (0, LANES - OUT))).astype(jnp.bfloat16)
    b2p = jnp.pad(b2, ((0, 0), (0, LANES - OUT)))
    scal = jnp.concatenate([jnp.stack([tau1, tau2]), bg])

    nsteps = B // BLK
    y, loss = pl.pallas_call(
        _moe_kernel,
        grid=(nsteps,),
        in_specs=[
            pl.BlockSpec(memory_space=pltpu.SMEM),
            pl.BlockSpec((BLK, IN), lambda i: (i, 0)),
            pl.BlockSpec((IN, HID), lambda i: (0, 0)),
            pl.BlockSpec((IN, HID), lambda i: (0, 0)),
            pl.BlockSpec((IN, LANES), lambda i: (0, 0)),
            pl.BlockSpec((E, HID), lambda i: (0, 0)),
            pl.BlockSpec((E * HID, LANES), lambda i: (0, 0)),
            pl.BlockSpec((E, LANES), lambda i: (0, 0)),
        ],
        out_specs=[
            pl.BlockSpec((BLK, OUT), lambda i: (i, 0)),
            pl.BlockSpec(block_shape=(1, 1), index_map=lambda i: (0, 0),
                         memory_space=pltpu.SMEM),
        ],
        out_shape=[
            jax.ShapeDtypeStruct((B, OUT), jnp.float32),
            jax.ShapeDtypeStruct((1, 1), jnp.float32),
        ],
        scratch_shapes=[pltpu.SMEM((2,), jnp.float32)],
    )(scal, xf, w1a, w1b, wgp, b1, w2c, b2p)

    return y, loss[0, 0]


def kernel(x, train, Wg, bg, W1, b1, W2, b2, tau1, tau2):
    del train
    return _moe(x, Wg, bg, W1, b1, W2, b2, tau1, tau2)


# speedup vs baseline: 1.0168x; 1.0168x over previous
import jax
import jax.numpy as jnp
from jax.experimental import pallas as pl
from jax.experimental.pallas import tpu as pltpu

B = 4096
IN = 3072
HID = 256
OUT = 10
E = 2
K = 2
LOSS_COEF = 0.01

BLK = 512
LANES = 128


def _moe_kernel(scal_ref, x_ref, w1a_ref, w1b_ref, wg_ref, b1_ref, w2_ref,
                b2_ref, y_ref, loss_ref, imp_ref):
    i = pl.program_id(0)
    nsteps = pl.num_programs(0)

    tau1 = scal_ref[0]
    tau2 = scal_ref[1]
    bg0 = scal_ref[2]
    bg1 = scal_ref[3]

    xb = x_ref[...].astype(jnp.bfloat16)
    pre0 = jnp.dot(xb, w1a_ref[...],
                   preferred_element_type=jnp.float32)
    pre1 = jnp.dot(xb, w1b_ref[...],
                   preferred_element_type=jnp.float32)
    gl = jnp.dot(xb, wg_ref[...],
                 preferred_element_type=jnp.float32)

    h0 = jnp.tanh(pre0 + b1_ref[0:1, :])
    h1 = jnp.tanh(pre1 + b1_ref[1:2, :])
    l0 = gl[:, 0:1] + bg0
    l1 = gl[:, 1:2] + bg1

    s0 = jax.nn.sigmoid(l0 - l1)
    s1 = jax.nn.sigmoid(l1 - l0)
    r0 = jax.nn.sigmoid((s1 - s0) / tau1)
    r1 = jax.nn.sigmoid((s0 - s1) / tau1)
    a0 = jax.nn.sigmoid((K + 0.5 - (1.0 + r0)) / tau2)
    a1 = jax.nn.sigmoid((K + 0.5 - (1.0 + r1)) / tau2)
    g0 = a0 * s0
    g1 = a1 * s1

    hs = jnp.concatenate([h0 * g0, h1 * g1], axis=1)
    out = jnp.dot(hs.astype(jnp.bfloat16), w2_ref[...],
                  preferred_element_type=jnp.float32)
    out = out + g0 * b2_ref[0:1, :] + g1 * b2_ref[1:2, :]
    y_ref[...] = out[:, :OUT]

    p0 = jnp.sum(g0)
    p1 = jnp.sum(g1)
    t0 = jnp.where(i == 0, 0.0, imp_ref[0]) + p0
    t1 = jnp.where(i == 0, 0.0, imp_ref[1]) + p1
    imp_ref[0] = t0
    imp_ref[1] = t1

    @pl.when(i == nsteps - 1)
    def _():
        m = (t0 + t1) * 0.5
        var = (t0 - m) ** 2 + (t1 - m) ** 2
        loss_ref[0, 0] = var / (m * m + 1e-10) * LOSS_COEF


@jax.jit
def _moe(x, Wg, bg, W1, b1, W2, b2, tau1, tau2):
    xf = x.reshape(B, IN)
    w1a = W1[0].astype(jnp.bfloat16)
    w1b = W1[1].astype(jnp.bfloat16)
    wgp = jnp.pad(Wg, ((0, 0), (0, LANES - E))).astype(jnp.bfloat16)
    w2c = jnp.pad(W2.reshape(E * HID, OUT),
                  ((0, 0), (0, LANES - OUT))).astype(jnp.bfloat16)
    b2p = jnp.pad(b2, ((0, 0), (0, LANES - OUT)))
    scal = jnp.concatenate([jnp.stack([tau1, tau2]), bg])

    nsteps = B // BLK
    y, loss = pl.pallas_call(
        _moe_kernel,
        grid=(nsteps,),
        in_specs=[
            pl.BlockSpec(memory_space=pltpu.SMEM),
            pl.BlockSpec((BLK, IN), lambda i: (i, 0)),
            pl.BlockSpec((IN, HID), lambda i: (0, 0)),
            pl.BlockSpec((IN, HID), lambda i: (0, 0)),
            pl.BlockSpec((IN, LANES), lambda i: (0, 0)),
            pl.BlockSpec((E, HID), lambda i: (0, 0)),
            pl.BlockSpec((E * HID, LANES), lambda i: (0, 0)),
            pl.BlockSpec((E, LANES), lambda i: (0, 0)),
        ],
        out_specs=[
            pl.BlockSpec((BLK, OUT), lambda i: (i, 0)),
            pl.BlockSpec(block_shape=(1, 1), index_map=lambda i: (0, 0),
                         memory_space=pltpu.SMEM),
        ],
        out_shape=[
            jax.ShapeDtypeStruct((B, OUT), jnp.float32),
            jax.ShapeDtypeStruct((1, 1), jnp.float32),
        ],
        scratch_shapes=[pltpu.SMEM((2,), jnp.float32)],
    )(scal, xf, w1a, w1b, wgp, b1, w2c, b2p)

    return y, loss[0, 0]


def kernel(x, train, Wg, bg, W1, b1, W2, b2, tau1, tau2):
    del train
    return _moe(x, Wg, bg, W1, b1, W2, b2, tau1, tau2)
